# baseline (speedup 1.0000x reference)
import functools

import jax
import jax.numpy as jnp
from jax import lax
from jax.experimental import pallas as pl
from jax.experimental.pallas import tpu as pltpu
from jax.experimental.pallas import tpu_sc as plsc

_N = 10000
_E = 320000
_ETOT = _N + _E
_NC = 2
_NS = 16
_NW = _NC * _NS
_K = 128
_CPW = (-(-_ETOT // (_NW * _K)) + 1) // 2 * 2
_EPAD = _NW * _K * _CPW
_WE = _CPW * _K
_RPS = 632
_NPAD = _RPS * _NS
_TW = 80
_AW = 80
_DW = 32


def _lrelu(v):
    return jnp.maximum(v, 0.2 * v)


def _make_edge_kernel(heads):
    outc = 64 // heads
    shift = outc.bit_length() - 1
    f32, i32 = jnp.float32, jnp.int32
    mesh = plsc.VectorSubcoreMesh(
        core_axis_name="c", subcore_axis_name="s",
        num_cores=_NC, num_subcores=_NS)

    @functools.partial(
        pl.kernel,
        out_type=jax.ShapeDtypeStruct((_NC, _NPAD, _AW), f32),
        mesh=mesh,
        compiler_params=pltpu.CompilerParams(use_tc_tiling_on_sc=False),
        scratch_types=[
            [pltpu.VMEM((_K,), i32)] * 2,
            [pltpu.VMEM((_K,), i32)] * 2,
            [pltpu.VMEM((_K, _TW), f32)] * 2,
            [pltpu.VMEM((_K, _DW), f32)] * 2,
            [pltpu.VMEM((_K, _AW), f32)] * 2,
            [pltpu.VMEM((_K,), i32)] * 2,
            pltpu.VMEM((_RPS // 8, _AW), f32),
            pltpu.VMEM_SHARED((_NPAD, _AW), f32),
            [pltpu.SemaphoreType.DMA] * 2,
            [pltpu.SemaphoreType.DMA] * 2,
            [pltpu.SemaphoreType.DMA] * 2,
        ],
    )
    def edge_kernel(table, adc, srcs, dsts, zeros, out,
                    idx_s, idx_d, rows_v, adc_v, msg_v, sidx, stage_v,
                    acc, gt, ga, ss):
        cid = lax.axis_index("c")
        sid = lax.axis_index("s")
        wid = sid * _NC + cid
        lanes = lax.broadcasted_iota(i32, (16,), 0)
        hd = jnp.bitwise_and(lanes, heads - 1)
        headvec = [lax.shift_right_logical(q * 16 + lanes, shift)
                   for q in range(4)]

        qr = _RPS // 8
        pltpu.sync_copy(zeros.at[pl.ds(0, qr)], stage_v)
        for r in range(8):
            pltpu.sync_copy(stage_v,
                            acc.at[pl.ds(sid * _RPS + r * qr, qr)])
        plsc.subcore_barrier()

        def fetch(t, b):
            base = (wid * _CPW + t) * _K
            pltpu.sync_copy(srcs.at[pl.ds(base, _K)], idx_s[b])
            pltpu.async_copy(table.at[idx_s[b]], rows_v[b], gt[b])
            pltpu.sync_copy(dsts.at[pl.ds(base, _K)], idx_d[b])
            pltpu.async_copy(adc.at[idx_d[b]], adc_v[b], ga[b])

        fetch(0, 0)
        fetch(1, 1)

        def pair(u, carry):
            for b in range(2):
                t = u * 2 + b
                pltpu.make_async_copy(
                    table.at[idx_s[b]], rows_v[b], gt[b]).wait()
                pltpu.make_async_copy(
                    adc.at[idx_d[b]], adc_v[b], ga[b]).wait()

                @pl.when(u > 0)
                def _():
                    pltpu.make_async_copy(
                        msg_v[b], acc.at[sidx[b]], ss[b]).wait()

                @plsc.parallel_loop(0, _K, unroll=4)
                def edge(e):
                    as_vec = rows_v[b][e, pl.ds(64, 16)]
                    ad_vec = adc_v[b][e, pl.ds(0, 16)]
                    c_vec = adc_v[b][e, pl.ds(16, 16)]
                    ex = jnp.exp(_lrelu(as_vec + ad_vec) - c_vec)
                    for q in range(4):
                        hvec = rows_v[b][e, pl.ds(q * 16, 16)]
                        if heads == 1:
                            exv = ex
                        else:
                            exv = ex.at[headvec[q]].get(
                                mode="promise_in_bounds")
                        msg_v[b][e, pl.ds(q * 16, 16)] = hvec * exv
                    if heads == 1:
                        msg_v[b][e, pl.ds(64, 16)] = ex
                    else:
                        msg_v[b][e, pl.ds(64, 16)] = ex.at[hd].get(
                            mode="promise_in_bounds")

                @plsc.parallel_loop(0, _K // 16, unroll=8)
                def cpidx(g):
                    sidx[b][pl.ds(g * 16, 16)] = idx_d[b][pl.ds(g * 16, 16)]
                pltpu.async_copy(msg_v[b], acc.at[sidx[b]], ss[b],
                                 add=True)

                @pl.when(t + 2 < _CPW)
                def _():
                    fetch(t + 2, b)
            return carry
        lax.fori_loop(0, _CPW // 2, pair, 0)

        for b in range(2):
            pltpu.make_async_copy(msg_v[b], acc.at[sidx[b]], ss[b]).wait()
        plsc.subcore_barrier()
        for r in range(8):
            pltpu.sync_copy(acc.at[pl.ds(sid * _RPS + r * qr, qr)], stage_v)
            pltpu.sync_copy(stage_v,
                            out.at[cid, pl.ds(sid * _RPS + r * qr, qr)])

    return edge_kernel


_edge8 = _make_edge_kernel(8)
_edge1 = _make_edge_kernel(1)


def _tc_pre1(x_ref, w1_ref, ams_ref, amd_ref, p_ref, table_ref, adc_ref):
    h = jnp.dot(x_ref[...], w1_ref[...], preferred_element_type=jnp.float32)
    a_s = jnp.dot(h, ams_ref[...], preferred_element_type=jnp.float32)
    a_d = jnp.dot(h, amd_ref[...], preferred_element_type=jnp.float32)
    gmax = jnp.max(a_s, axis=0, keepdims=True)
    p = p_ref[...]
    table_ref[:, 0:64] = h
    table_ref[:, 64:80] = jnp.dot(a_s, p, preferred_element_type=jnp.float32)
    adc_ref[:, 0:16] = jnp.dot(a_d, p, preferred_element_type=jnp.float32)
    adc_ref[:, 16:32] = jnp.dot(_lrelu(gmax + a_d), p,
                                preferred_element_type=jnp.float32)


def _tc_mid(acc_ref, b1_ref, w2_ref, ams_ref, amd_ref, rep_ref, p_ref,
            table_ref, adc_ref):
    s = acc_ref[0] + acc_ref[1]
    denom = jnp.dot(s[:, 64:72], rep_ref[...],
                    preferred_element_type=jnp.float32)
    out1 = s[:, 0:64] / (denom + 1e-16) + b1_ref[...]
    h2 = jnp.dot(out1, w2_ref[...], preferred_element_type=jnp.float32)
    a_s = jnp.dot(h2, ams_ref[...], preferred_element_type=jnp.float32)
    a_d = jnp.dot(h2, amd_ref[...], preferred_element_type=jnp.float32)
    gmax = jnp.max(a_s, axis=0, keepdims=True)
    p = p_ref[...]
    table_ref[:, 0:64] = h2
    table_ref[:, 64:80] = jnp.dot(a_s, p, preferred_element_type=jnp.float32)
    adc_ref[:, 0:16] = jnp.dot(a_d, p, preferred_element_type=jnp.float32)
    adc_ref[:, 16:32] = jnp.dot(_lrelu(gmax + a_d), p,
                                preferred_element_type=jnp.float32)


def _tc_post(acc_ref, b2_ref, rep_ref, out_ref):
    s = acc_ref[0] + acc_ref[1]
    denom = jnp.dot(s[:, 64:72], rep_ref[...],
                    preferred_element_type=jnp.float32)
    out_ref[...] = s[:, 0:64] / (denom + 1e-16) + b2_ref[...]


def kernel(x, edge_index, W1, a_src1, a_dst1, b1, W2, a_src2, a_dst2, b2):
    f32, i32 = jnp.float32, jnp.int32
    loops = jnp.arange(_N, dtype=i32)
    padi = jnp.full((_EPAD - _ETOT,), _N, dtype=i32)
    src = jnp.concatenate([edge_index[0].astype(i32), loops, padi])
    dst = jnp.concatenate([edge_index[1].astype(i32), loops, padi])
    xpad = jnp.pad(x, ((0, _NPAD - _N), (0, 0)))
    zeros = jnp.zeros((_NPAD, _AW), f32)

    eye8 = jnp.eye(8, dtype=f32)
    expand = jnp.repeat(eye8, 8, axis=0)
    am1s = a_src1.reshape(64, 1) * expand
    am1d = a_dst1.reshape(64, 1) * expand
    rep = jnp.repeat(eye8, 8, axis=1)
    am2s = jnp.pad(a_src2.reshape(64, 1), ((0, 0), (0, 7)))
    am2d = jnp.pad(a_dst2.reshape(64, 1), ((0, 0), (0, 7)))

    p1 = jnp.pad(eye8, ((0, 0), (0, 8)))
    p2 = jnp.zeros((8, 16), f32).at[0, :].set(1.0)
    table1, adc1 = pl.pallas_call(
        _tc_pre1,
        out_shape=[jax.ShapeDtypeStruct((_NPAD, _TW), f32),
                   jax.ShapeDtypeStruct((_NPAD, _DW), f32)],
    )(xpad, W1, am1s, am1d, p1)

    acc1 = _edge8(table1, adc1, src, dst, zeros)

    table2, adc2 = pl.pallas_call(
        _tc_mid,
        out_shape=[jax.ShapeDtypeStruct((_NPAD, _TW), f32),
                   jax.ShapeDtypeStruct((_NPAD, _DW), f32)],
    )(acc1, b1.reshape(1, 64), W2, am2s, am2d, rep, p2)

    acc2 = _edge1(table2, adc2, src, dst, zeros)

    outp = pl.pallas_call(
        _tc_post,
        out_shape=jax.ShapeDtypeStruct((_NPAD, 64), f32),
    )(acc2, b2.reshape(1, 64), rep)
    return outp[:_N]


# speedup vs baseline: 1.0242x; 1.0242x over previous
import functools

import jax
import jax.numpy as jnp
from jax import lax
from jax.experimental import pallas as pl
from jax.experimental.pallas import tpu as pltpu
from jax.experimental.pallas import tpu_sc as plsc

_N = 10000
_E = 320000
_ETOT = _N + _E
_NC = 2
_NS = 16
_NW = _NC * _NS
_K = 128
_CPW = (-(-_ETOT // (_NW * _K)) + 1) // 2 * 2
_EPAD = _NW * _K * _CPW
_WE = _CPW * _K
_RPS = 632
_NPAD = _RPS * _NS
_TW = 80
_AW = 80
_DW = 32


def _lrelu(v):
    return jnp.maximum(v, 0.2 * v)


def _make_edge_kernel(heads):
    outc = 64 // heads
    shift = outc.bit_length() - 1
    f32, i32 = jnp.float32, jnp.int32
    mesh = plsc.VectorSubcoreMesh(
        core_axis_name="c", subcore_axis_name="s",
        num_cores=_NC, num_subcores=_NS)

    @functools.partial(
        pl.kernel,
        out_type=jax.ShapeDtypeStruct((_NC, _NPAD, _AW), f32),
        mesh=mesh,
        compiler_params=pltpu.CompilerParams(use_tc_tiling_on_sc=False),
        scratch_types=[
            [pltpu.VMEM((_K,), i32)] * 2,
            pltpu.VMEM((_WE,), i32),
            [pltpu.VMEM((_K, _TW), f32)] * 2,
            [pltpu.VMEM((_K, _DW), f32)] * 2,
            [pltpu.VMEM((_K, _AW), f32)] * 2,
            [pltpu.VMEM((_K,), i32)] * 2,
            pltpu.VMEM((_RPS // 8, _AW), f32),
            pltpu.VMEM_SHARED((_NPAD, _AW), f32),
            [pltpu.SemaphoreType.DMA] * 2,
            [pltpu.SemaphoreType.DMA] * 2,
            [pltpu.SemaphoreType.DMA] * 2,
        ],
    )
    def edge_kernel(table, adc, srcs, dsts, zeros, out,
                    idx_s, all_d, rows_v, adc_v, msg_v, sidx, stage_v,
                    acc, gt, ga, ss):
        cid = lax.axis_index("c")
        sid = lax.axis_index("s")
        wid = sid * _NC + cid
        lanes = lax.broadcasted_iota(i32, (16,), 0)
        hd = jnp.bitwise_and(lanes, heads - 1)
        headvec = [lax.shift_right_logical(q * 16 + lanes, shift)
                   for q in range(4)]

        pltpu.sync_copy(dsts.at[pl.ds(wid * _WE, _WE)], all_d)
        qr = _RPS // 8
        pltpu.sync_copy(zeros.at[pl.ds(0, qr)], stage_v)
        for r in range(8):
            pltpu.sync_copy(stage_v,
                            acc.at[pl.ds(sid * _RPS + r * qr, qr)])
        plsc.subcore_barrier()

        def fetch(t, b):
            base = (wid * _CPW + t) * _K
            pltpu.sync_copy(srcs.at[pl.ds(base, _K)], idx_s[b])
            pltpu.async_copy(table.at[idx_s[b]], rows_v[b], gt[b])
            pltpu.async_copy(adc.at[all_d.at[pl.ds(t * _K, _K)]],
                             adc_v[b], ga[b])

        fetch(0, 0)
        fetch(1, 1)

        def pair(u, carry):
            for b in range(2):
                t = u * 2 + b
                pltpu.make_async_copy(
                    table.at[idx_s[b]], rows_v[b], gt[b]).wait()
                pltpu.make_async_copy(
                    adc.at[all_d.at[pl.ds(t * _K, _K)]],
                    adc_v[b], ga[b]).wait()

                @pl.when(u > 0)
                def _():
                    pltpu.make_async_copy(
                        msg_v[b], acc.at[sidx[b]], ss[b]).wait()

                @plsc.parallel_loop(0, _K, unroll=4)
                def edge(e):
                    as_vec = rows_v[b][e, pl.ds(64, 16)]
                    ad_vec = adc_v[b][e, pl.ds(0, 16)]
                    c_vec = adc_v[b][e, pl.ds(16, 16)]
                    ex = jnp.exp(_lrelu(as_vec + ad_vec) - c_vec)
                    for q in range(4):
                        hvec = rows_v[b][e, pl.ds(q * 16, 16)]
                        if heads == 1:
                            exv = ex
                        else:
                            exv = ex.at[headvec[q]].get(
                                mode="promise_in_bounds")
                        msg_v[b][e, pl.ds(q * 16, 16)] = hvec * exv
                    if heads == 1:
                        msg_v[b][e, pl.ds(64, 16)] = ex
                    else:
                        msg_v[b][e, pl.ds(64, 16)] = ex.at[hd].get(
                            mode="promise_in_bounds")

                @plsc.parallel_loop(0, _K // 16, unroll=8)
                def cpidx(g):
                    sidx[b][pl.ds(g * 16, 16)] = all_d[
                        pl.ds(t * _K + g * 16, 16)]
                pltpu.async_copy(msg_v[b], acc.at[sidx[b]], ss[b],
                                 add=True)

                @pl.when(t + 2 < _CPW)
                def _():
                    fetch(t + 2, b)
            return carry
        lax.fori_loop(0, _CPW // 2, pair, 0)

        for b in range(2):
            pltpu.make_async_copy(msg_v[b], acc.at[sidx[b]], ss[b]).wait()
        plsc.subcore_barrier()
        for r in range(8):
            pltpu.sync_copy(acc.at[pl.ds(sid * _RPS + r * qr, qr)], stage_v)
            pltpu.sync_copy(stage_v,
                            out.at[cid, pl.ds(sid * _RPS + r * qr, qr)])

    return edge_kernel


_edge8 = _make_edge_kernel(8)
_edge1 = _make_edge_kernel(1)


def _tc_pre1(x_ref, w1_ref, ams_ref, amd_ref, p_ref, table_ref, adc_ref):
    h = jnp.dot(x_ref[...], w1_ref[...], preferred_element_type=jnp.float32)
    a_s = jnp.dot(h, ams_ref[...], preferred_element_type=jnp.float32)
    a_d = jnp.dot(h, amd_ref[...], preferred_element_type=jnp.float32)
    gmax = jnp.max(a_s, axis=0, keepdims=True)
    p = p_ref[...]
    table_ref[:, 0:64] = h
    table_ref[:, 64:80] = jnp.dot(a_s, p, preferred_element_type=jnp.float32)
    adc_ref[:, 0:16] = jnp.dot(a_d, p, preferred_element_type=jnp.float32)
    adc_ref[:, 16:32] = jnp.dot(_lrelu(gmax + a_d), p,
                                preferred_element_type=jnp.float32)


def _tc_mid(acc_ref, b1_ref, w2_ref, ams_ref, amd_ref, rep_ref, p_ref,
            table_ref, adc_ref):
    s = acc_ref[0] + acc_ref[1]
    denom = jnp.dot(s[:, 64:72], rep_ref[...],
                    preferred_element_type=jnp.float32)
    out1 = s[:, 0:64] / (denom + 1e-16) + b1_ref[...]
    h2 = jnp.dot(out1, w2_ref[...], preferred_element_type=jnp.float32)
    a_s = jnp.dot(h2, ams_ref[...], preferred_element_type=jnp.float32)
    a_d = jnp.dot(h2, amd_ref[...], preferred_element_type=jnp.float32)
    gmax = jnp.max(a_s, axis=0, keepdims=True)
    p = p_ref[...]
    table_ref[:, 0:64] = h2
    table_ref[:, 64:80] = jnp.dot(a_s, p, preferred_element_type=jnp.float32)
    adc_ref[:, 0:16] = jnp.dot(a_d, p, preferred_element_type=jnp.float32)
    adc_ref[:, 16:32] = jnp.dot(_lrelu(gmax + a_d), p,
                                preferred_element_type=jnp.float32)


def _tc_post(acc_ref, b2_ref, rep_ref, out_ref):
    s = acc_ref[0] + acc_ref[1]
    denom = jnp.dot(s[:, 64:72], rep_ref[...],
                    preferred_element_type=jnp.float32)
    out_ref[...] = s[:, 0:64] / (denom + 1e-16) + b2_ref[...]


def kernel(x, edge_index, W1, a_src1, a_dst1, b1, W2, a_src2, a_dst2, b2):
    f32, i32 = jnp.float32, jnp.int32
    loops = jnp.arange(_N, dtype=i32)
    padi = jnp.full((_EPAD - _ETOT,), _N, dtype=i32)
    src = jnp.concatenate([edge_index[0].astype(i32), loops, padi])
    dst = jnp.concatenate([edge_index[1].astype(i32), loops, padi])
    xpad = jnp.pad(x, ((0, _NPAD - _N), (0, 0)))
    zeros = jnp.zeros((_NPAD, _AW), f32)

    eye8 = jnp.eye(8, dtype=f32)
    expand = jnp.repeat(eye8, 8, axis=0)
    am1s = a_src1.reshape(64, 1) * expand
    am1d = a_dst1.reshape(64, 1) * expand
    rep = jnp.repeat(eye8, 8, axis=1)
    am2s = jnp.pad(a_src2.reshape(64, 1), ((0, 0), (0, 7)))
    am2d = jnp.pad(a_dst2.reshape(64, 1), ((0, 0), (0, 7)))

    p1 = jnp.pad(eye8, ((0, 0), (0, 8)))
    p2 = jnp.zeros((8, 16), f32).at[0, :].set(1.0)
    table1, adc1 = pl.pallas_call(
        _tc_pre1,
        out_shape=[jax.ShapeDtypeStruct((_NPAD, _TW), f32),
                   jax.ShapeDtypeStruct((_NPAD, _DW), f32)],
    )(xpad, W1, am1s, am1d, p1)

    acc1 = _edge8(table1, adc1, src, dst, zeros)

    table2, adc2 = pl.pallas_call(
        _tc_mid,
        out_shape=[jax.ShapeDtypeStruct((_NPAD, _TW), f32),
                   jax.ShapeDtypeStruct((_NPAD, _DW), f32)],
    )(acc1, b1.reshape(1, 64), W2, am2s, am2d, rep, p2)

    acc2 = _edge1(table2, adc2, src, dst, zeros)

    outp = pl.pallas_call(
        _tc_post,
        out_shape=jax.ShapeDtypeStruct((_NPAD, 64), f32),
    )(acc2, b2.reshape(1, 64), rep)
    return outp[:_N]
